# baseline (speedup 1.0000x reference)
import functools

import jax
import jax.numpy as jnp
from jax import lax
from jax.experimental import pallas as pl
from jax.experimental.pallas import tpu as pltpu
from jax.experimental.pallas import tpu_sc as plsc

N = 50000
E = 800000
HID = 64
G = 128

NP = 50176
ER = 819200
EPT = 25600
RCAP = 26624
QCAP = 26656
HALF = 25000
ACC_ROWS = 25088
TRASH = 25080
TPS = 1568
DTRASH = 1 << 20


def _sc_router():
    mesh = plsc.VectorSubcoreMesh(core_axis_name="c", subcore_axis_name="s")

    @functools.partial(
        pl.kernel,
        out_type=[
            jax.ShapeDtypeStruct((2, 32 * RCAP), jnp.int32),
            jax.ShapeDtypeStruct((2, 32 * RCAP), jnp.int32),
            jax.ShapeDtypeStruct((2, 32, 16), jnp.int32),
        ],
        mesh=mesh,
        compiler_params=pltpu.CompilerParams(use_tc_tiling_on_sc=False,
                                             needs_layout_passes=False),
        scratch_types=[
            pltpu.VMEM((1024,), jnp.int32),
            pltpu.VMEM((1024,), jnp.int32),
            pltpu.VMEM((QCAP,), jnp.int32),
            pltpu.VMEM((QCAP,), jnp.int32),
            pltpu.VMEM((QCAP,), jnp.int32),
            pltpu.VMEM((QCAP,), jnp.int32),
            pltpu.VMEM((16,), jnp.int32),
        ],
    )
    def k(src_hbm, dst_hbm, qs_out, qd_out, cnt_out,
          sstg, dstg, qsa, qda, qsb, qdb, cntv):
        c = lax.axis_index("c")
        s = lax.axis_index("s")
        w = c * 16 + s
        e0 = w * EPT
        lanes = lax.iota(jnp.int32, 16)
        junk = jnp.full((16,), RCAP + 16, jnp.int32) + lanes

        def chunk(i, carry):
            ptr_a, ptr_b = carry
            off = e0 + i * 1024
            pltpu.sync_copy(src_hbm.at[pl.ds(off, 1024)], sstg)
            pltpu.sync_copy(dst_hbm.at[pl.ds(off, 1024)], dstg)
            for t in range(64):
                sv = sstg[pl.ds(t * 16, 16)]
                dv = dstg[pl.ds(t * 16, 16)]
                m_a = dv < HALF
                mi = jnp.where(m_a, 1, 0)
                cs = plsc.cumsum(mi)
                idx_a = jnp.where(m_a, ptr_a + cs - 1, junk)
                idx_b = jnp.where(m_a, junk, ptr_b + (lanes - cs))
                plsc.store_scatter(qsa, [idx_a], sv)
                plsc.store_scatter(qda, [idx_a], dv)
                plsc.store_scatter(qsb, [idx_b], sv)
                plsc.store_scatter(qdb, [idx_b], dv)
                cnt = jnp.sum(mi)
                ptr_a = ptr_a + cnt
                ptr_b = ptr_b + (16 - cnt)
            return ptr_a, ptr_b

        ptr_a, ptr_b = lax.fori_loop(0, EPT // 1024, chunk, (0, 0))

        tr_s = jnp.zeros((16,), jnp.int32)
        tr_d = jnp.full((16,), DTRASH, jnp.int32)

        def pad_queue(qs_ref, qd_ref, ptr):
            bound = ((ptr + 2047) // 2048) * 2048
            trips = (bound - ptr + 15) // 16

            def pb(_, pr):
                idx = pr + lanes
                plsc.store_scatter(qs_ref, [idx], tr_s)
                plsc.store_scatter(qd_ref, [idx], tr_d)
                return pr + 16

            lax.fori_loop(0, trips, pb, ptr)
            return bound // 2048

        np_a = pad_queue(qsa, qda, ptr_a)
        np_b = pad_queue(qsb, qdb, ptr_b)

        cntv[...] = jnp.where(lanes == 0, np_a, 0)
        pltpu.sync_copy(cntv, cnt_out.at[0, w])
        cntv[...] = jnp.where(lanes == 0, np_b, 0)
        pltpu.sync_copy(cntv, cnt_out.at[1, w])
        pltpu.sync_copy(qsa.at[pl.ds(0, RCAP)],
                        qs_out.at[0, pl.ds(w * RCAP, RCAP)])
        pltpu.sync_copy(qda.at[pl.ds(0, RCAP)],
                        qd_out.at[0, pl.ds(w * RCAP, RCAP)])
        pltpu.sync_copy(qsb.at[pl.ds(0, RCAP)],
                        qs_out.at[1, pl.ds(w * RCAP, RCAP)])
        pltpu.sync_copy(qdb.at[pl.ds(0, RCAP)],
                        qd_out.at[1, pl.ds(w * RCAP, RCAP)])

    return k


_router = _sc_router()


def _sc_aggr(d_feat):
    mesh = plsc.VectorSubcoreMesh(core_axis_name="c", subcore_axis_name="s")

    @functools.partial(
        pl.kernel,
        out_type=jax.ShapeDtypeStruct((NP, d_feat), jnp.float32),
        mesh=mesh,
        compiler_params=pltpu.CompilerParams(use_tc_tiling_on_sc=False),
        scratch_types=[
            pltpu.VMEM((16, 128), jnp.int32),
            pltpu.VMEM((16, 128), jnp.int32),
            pltpu.VMEM((16, 128), jnp.int32),
            pltpu.VMEM((128, d_feat), jnp.float32),
            pltpu.VMEM((2, 16), jnp.int32),
            pltpu.VMEM_SHARED((ACC_ROWS, d_feat), jnp.float32),
            pltpu.SemaphoreType.DMA,
        ],
    )
    def k(qs_hbm, qd_hbm, cnt_hbm, zeros_hbm, table, out_hbm,
          srcstg, dststg, idxb, buf, cntw, acc, sem):
        c = lax.axis_index("c")
        s = lax.axis_index("s")
        lo = c * HALF
        pltpu.sync_copy(cnt_hbm.at[c, pl.ds(2 * s, 2)], cntw)
        pltpu.sync_copy(zeros_hbm, acc.at[pl.ds(s * TPS, TPS)])
        plsc.subcore_barrier()

        def pair(i, base):
            rb = base + i * 16
            pltpu.sync_copy(qs_hbm.at[c, pl.ds(rb, 16)], srcstg)
            pltpu.sync_copy(qd_hbm.at[c, pl.ds(rb, 16)], dststg)
            for r in range(16):
                for t in range(8):
                    d = dststg[r, pl.ds(t * 16, 16)]
                    dl = d - lo
                    ok = (dl >= 0) & (dl < HALF)
                    idxb[r, pl.ds(t * 16, 16)] = jnp.where(ok, dl, TRASH)
            for r in range(16):
                pltpu.async_copy(table.at[srcstg.at[r]], buf, sem).wait()
                pltpu.sync_copy(buf, acc.at[idxb.at[r]], add=True)
            return base

        for r in (0, 1):
            region = 2 * s + r
            npr = cntw[r, pl.ds(0, 16)][0]
            lax.fori_loop(0, npr, pair, region * (RCAP // 128))
        plsc.subcore_barrier()
        @pl.when(s < 15)
        def _():
            pltpu.sync_copy(acc.at[pl.ds(s * TPS, TPS)],
                            out_hbm.at[pl.ds(lo + s * TPS, TPS)])

        @pl.when(s == 15)
        def _():
            pltpu.sync_copy(acc.at[pl.ds(15 * TPS, HALF - 15 * TPS)],
                            out_hbm.at[pl.ds(lo + 15 * TPS,
                                             HALF - 15 * TPS)])

    return k


_sc_aggr16 = _sc_aggr(16)
_sc_aggr64 = _sc_aggr(HID)


BM = 1568


def _tc1_body(x_ref, a_ref, w1_ref, b1_ref, w2_ref, b2_ref, o_ref):
    z = x_ref[...] + a_ref[...]
    h = jnp.maximum(
        jnp.dot(z, w1_ref[...], preferred_element_type=jnp.float32)
        + b1_ref[...], 0.0)
    o_ref[...] = jnp.maximum(
        jnp.dot(h, w2_ref[...], preferred_element_type=jnp.float32)
        + b2_ref[...], 0.0)


def _tc1(x_pad, aggr1, W1p, b1, W2, b2):
    grid = NP // BM
    return pl.pallas_call(
        _tc1_body,
        grid=(grid,),
        in_specs=[
            pl.BlockSpec((BM, 16), lambda i: (i, 0)),
            pl.BlockSpec((BM, 16), lambda i: (i, 0)),
            pl.BlockSpec((16, HID), lambda i: (0, 0)),
            pl.BlockSpec((1, HID), lambda i: (0, 0)),
            pl.BlockSpec((HID, HID), lambda i: (0, 0)),
            pl.BlockSpec((1, HID), lambda i: (0, 0)),
        ],
        out_specs=pl.BlockSpec((BM, HID), lambda i: (i, 0)),
        out_shape=jax.ShapeDtypeStruct((NP, HID), jnp.float32),
        compiler_params=pltpu.CompilerParams(
            dimension_semantics=("arbitrary",)),
    )(x_pad, aggr1, W1p, b1, W2, b2)


def _tc2_body(h_ref, a_ref, b_ref, w3_ref, b3_ref, w4_ref, b4_ref,
              pool_ref, out_ref):
    i = pl.program_id(0)
    nblk = pl.num_programs(0)
    z = h_ref[...] + a_ref[...]
    t = jnp.maximum(
        jnp.dot(z, w3_ref[...], preferred_element_type=jnp.float32)
        + b3_ref[...], 0.0)
    h2 = jnp.maximum(
        jnp.dot(t, w4_ref[...], preferred_element_type=jnp.float32)
        + b4_ref[...], 0.0)
    bidx = b_ref[...]
    valid = bidx < G
    h2 = jnp.where(valid, h2, 0.0)
    onehot = (bidx == lax.broadcasted_iota(jnp.int32, (BM, G), 1))
    onehot = onehot.astype(jnp.float32)
    ones_col = jnp.where(valid, 1.0, 0.0)
    hc = jnp.concatenate(
        [h2, ones_col, jnp.zeros((BM, 15), jnp.float32)], axis=1)
    contrib = lax.dot_general(
        onehot, hc, (((0,), (0,)), ((), ())),
        preferred_element_type=jnp.float32)

    @pl.when(i == 0)
    def _():
        pool_ref[...] = jnp.zeros_like(pool_ref)

    pool_ref[...] += contrib

    @pl.when(i == nblk - 1)
    def _():
        p = pool_ref[...]
        cnt = jnp.maximum(p[:, HID:HID + 1], 1.0)
        out_ref[...] = p[:, :HID] / cnt


def _tc2(h1, aggr2, batch2d, W3, b3, W4, b4):
    grid = NP // BM
    _, out = pl.pallas_call(
        _tc2_body,
        grid=(grid,),
        in_specs=[
            pl.BlockSpec((BM, HID), lambda i: (i, 0)),
            pl.BlockSpec((BM, HID), lambda i: (i, 0)),
            pl.BlockSpec((BM, 1), lambda i: (i, 0)),
            pl.BlockSpec((HID, HID), lambda i: (0, 0)),
            pl.BlockSpec((1, HID), lambda i: (0, 0)),
            pl.BlockSpec((HID, HID), lambda i: (0, 0)),
            pl.BlockSpec((1, HID), lambda i: (0, 0)),
        ],
        out_specs=[
            pl.BlockSpec((G, HID + 16), lambda i: (0, 0)),
            pl.BlockSpec((G, HID), lambda i: (0, 0)),
        ],
        out_shape=[
            jax.ShapeDtypeStruct((G, HID + 16), jnp.float32),
            jax.ShapeDtypeStruct((G, HID), jnp.float32),
        ],
        compiler_params=pltpu.CompilerParams(
            dimension_semantics=("arbitrary",)),
    )(h1, aggr2, batch2d, W3, b3, W4, b4)
    return out


def kernel(x, edge_index, batch, W1, b1, W2, b2, W3, b3, W4, b4):
    ei = jnp.asarray(edge_index, jnp.int32)
    src = jnp.concatenate([ei[0], jnp.zeros((ER - E,), jnp.int32)])
    dst = jnp.concatenate([ei[1], jnp.full((ER - E,), DTRASH, jnp.int32)])

    x_pad = jnp.pad(x, ((0, NP - N), (0, 16 - x.shape[1])))
    W1p = jnp.pad(W1, ((0, 16 - W1.shape[0]), (0, 0)))
    batch2d = jnp.concatenate(
        [jnp.asarray(batch, jnp.int32),
         jnp.full((NP - N,), G, jnp.int32)]).reshape(NP, 1)

    z16 = jnp.zeros((TPS, 16), jnp.float32)
    z64 = jnp.zeros((TPS, HID), jnp.float32)

    qs, qd, cnt = _router(src, dst)
    qs = qs.reshape(2, (32 * RCAP) // 128, 128)
    qd = qd.reshape(2, (32 * RCAP) // 128, 128)
    aggr1 = _sc_aggr16(qs, qd, cnt, z16, x_pad)
    h1 = _tc1(x_pad, aggr1, W1p, b1.reshape(1, HID), W2, b2.reshape(1, HID))
    aggr2 = _sc_aggr64(qs, qd, cnt, z64, h1)
    return _tc2(h1, aggr2, batch2d, W3, b3.reshape(1, HID),
                W4, b4.reshape(1, HID))


# speedup vs baseline: 1.8166x; 1.8166x over previous
import functools

import jax
import jax.numpy as jnp
from jax import lax
from jax.experimental import pallas as pl
from jax.experimental.pallas import tpu as pltpu
from jax.experimental.pallas import tpu_sc as plsc

N = 50000
E = 800000
HID = 64
G = 128

NP = 50176
EP = 819200
EROWS = EP // 128
ROWS_PER_TILE = EROWS // 16
HALF = 25000
ACC_ROWS = 25088
TRASH = 25080
TPS = 1568


def _sc_aggr(d_feat):
    mesh = plsc.VectorSubcoreMesh(core_axis_name="c", subcore_axis_name="s")

    @functools.partial(
        pl.kernel,
        out_type=jax.ShapeDtypeStruct((NP, d_feat), jnp.float32),
        mesh=mesh,
        compiler_params=pltpu.CompilerParams(use_tc_tiling_on_sc=False),
        scratch_types=[
            pltpu.VMEM((8, 128), jnp.int32),
            pltpu.VMEM((8, 128), jnp.int32),
            pltpu.VMEM((8, 128), jnp.int32),
            pltpu.VMEM((128, d_feat), jnp.float32),
            pltpu.VMEM_SHARED((ACC_ROWS, d_feat), jnp.float32),
            pltpu.SemaphoreType.DMA,
        ],
    )
    def k(src_hbm, dst_hbm, zeros_hbm, vals_hbm, out_hbm,
          src_v, dst_v, idx_v, rows_v, acc, sem):
        c = lax.axis_index("c")
        s = lax.axis_index("s")
        pltpu.sync_copy(zeros_hbm, acc.at[pl.ds(s * TPS, TPS)])
        plsc.subcore_barrier()
        lo = c * HALF

        def body(i, carry):
            r0 = s * ROWS_PER_TILE + i * 8
            pltpu.sync_copy(src_hbm.at[pl.ds(r0, 8)], src_v)
            pltpu.sync_copy(dst_hbm.at[pl.ds(r0, 8)], dst_v)
            for j in range(8):
                for t in range(8):
                    d = dst_v[j, pl.ds(t * 16, 16)]
                    dl = d - lo
                    ok = (dl >= 0) & (dl < HALF)
                    idx_v[j, pl.ds(t * 16, 16)] = jnp.where(ok, dl, TRASH)
            for j in range(8):
                pltpu.async_copy(vals_hbm.at[src_v.at[j]], rows_v, sem).wait()
                pltpu.sync_copy(rows_v, acc.at[idx_v.at[j]], add=True)
            return carry

        lax.fori_loop(0, ROWS_PER_TILE // 8, body, 0)
        plsc.subcore_barrier()
        @pl.when(s < 15)
        def _():
            pltpu.sync_copy(acc.at[pl.ds(s * TPS, TPS)],
                            out_hbm.at[pl.ds(lo + s * TPS, TPS)])

        @pl.when(s == 15)
        def _():
            pltpu.sync_copy(acc.at[pl.ds(15 * TPS, HALF - 15 * TPS)],
                            out_hbm.at[pl.ds(lo + 15 * TPS, HALF - 15 * TPS)])

    return k


_sc_aggr16 = _sc_aggr(16)
_sc_aggr64 = _sc_aggr(HID)


BM = 1568


def _tc1_body(x_ref, a_ref, w1_ref, b1_ref, w2_ref, b2_ref, o_ref):
    z = x_ref[...] + a_ref[...]
    h = jnp.maximum(
        jnp.dot(z, w1_ref[...], preferred_element_type=jnp.float32)
        + b1_ref[...], 0.0)
    o_ref[...] = jnp.maximum(
        jnp.dot(h, w2_ref[...], preferred_element_type=jnp.float32)
        + b2_ref[...], 0.0)


def _tc1(x_pad, aggr1, W1p, b1, W2, b2):
    grid = NP // BM
    return pl.pallas_call(
        _tc1_body,
        grid=(grid,),
        in_specs=[
            pl.BlockSpec((BM, 16), lambda i: (i, 0)),
            pl.BlockSpec((BM, 16), lambda i: (i, 0)),
            pl.BlockSpec((16, HID), lambda i: (0, 0)),
            pl.BlockSpec((1, HID), lambda i: (0, 0)),
            pl.BlockSpec((HID, HID), lambda i: (0, 0)),
            pl.BlockSpec((1, HID), lambda i: (0, 0)),
        ],
        out_specs=pl.BlockSpec((BM, HID), lambda i: (i, 0)),
        out_shape=jax.ShapeDtypeStruct((NP, HID), jnp.float32),
        compiler_params=pltpu.CompilerParams(
            dimension_semantics=("arbitrary",)),
    )(x_pad, aggr1, W1p, b1, W2, b2)


def _tc2_body(h_ref, a_ref, b_ref, w3_ref, b3_ref, w4_ref, b4_ref,
              pool_ref, out_ref):
    i = pl.program_id(0)
    nblk = pl.num_programs(0)
    z = h_ref[...] + a_ref[...]
    t = jnp.maximum(
        jnp.dot(z, w3_ref[...], preferred_element_type=jnp.float32)
        + b3_ref[...], 0.0)
    h2 = jnp.maximum(
        jnp.dot(t, w4_ref[...], preferred_element_type=jnp.float32)
        + b4_ref[...], 0.0)
    bidx = b_ref[...]
    valid = bidx < G
    h2 = jnp.where(valid, h2, 0.0)
    onehot = (bidx == lax.broadcasted_iota(jnp.int32, (BM, G), 1))
    onehot = onehot.astype(jnp.float32)
    ones_col = jnp.where(valid, 1.0, 0.0)
    hc = jnp.concatenate(
        [h2, ones_col, jnp.zeros((BM, 15), jnp.float32)], axis=1)
    contrib = lax.dot_general(
        onehot, hc, (((0,), (0,)), ((), ())),
        preferred_element_type=jnp.float32)

    @pl.when(i == 0)
    def _():
        pool_ref[...] = jnp.zeros_like(pool_ref)

    pool_ref[...] += contrib

    @pl.when(i == nblk - 1)
    def _():
        p = pool_ref[...]
        cnt = jnp.maximum(p[:, HID:HID + 1], 1.0)
        out_ref[...] = p[:, :HID] / cnt


def _tc2(h1, aggr2, batch2d, W3, b3, W4, b4):
    grid = NP // BM
    _, out = pl.pallas_call(
        _tc2_body,
        grid=(grid,),
        in_specs=[
            pl.BlockSpec((BM, HID), lambda i: (i, 0)),
            pl.BlockSpec((BM, HID), lambda i: (i, 0)),
            pl.BlockSpec((BM, 1), lambda i: (i, 0)),
            pl.BlockSpec((HID, HID), lambda i: (0, 0)),
            pl.BlockSpec((1, HID), lambda i: (0, 0)),
            pl.BlockSpec((HID, HID), lambda i: (0, 0)),
            pl.BlockSpec((1, HID), lambda i: (0, 0)),
        ],
        out_specs=[
            pl.BlockSpec((G, HID + 16), lambda i: (0, 0)),
            pl.BlockSpec((G, HID), lambda i: (0, 0)),
        ],
        out_shape=[
            jax.ShapeDtypeStruct((G, HID + 16), jnp.float32),
            jax.ShapeDtypeStruct((G, HID), jnp.float32),
        ],
        compiler_params=pltpu.CompilerParams(
            dimension_semantics=("arbitrary",)),
    )(h1, aggr2, batch2d, W3, b3, W4, b4)
    return out


def kernel(x, edge_index, batch, W1, b1, W2, b2, W3, b3, W4, b4):
    ei = jnp.asarray(edge_index, jnp.int32)
    src = jnp.concatenate([ei[0], jnp.zeros((EP - E,), jnp.int32)])
    dst = jnp.concatenate([ei[1], jnp.full((EP - E,), 1 << 20, jnp.int32)])
    src2d = src.reshape(EROWS, 128)
    dst2d = dst.reshape(EROWS, 128)

    x_pad = jnp.pad(x, ((0, NP - N), (0, 16 - x.shape[1])))
    W1p = jnp.pad(W1, ((0, 16 - W1.shape[0]), (0, 0)))
    batch2d = jnp.concatenate(
        [jnp.asarray(batch, jnp.int32),
         jnp.full((NP - N,), G, jnp.int32)]).reshape(NP, 1)

    z16 = jnp.zeros((TPS, 16), jnp.float32)
    z64 = jnp.zeros((TPS, HID), jnp.float32)

    aggr1 = _sc_aggr16(src2d, dst2d, z16, x_pad)
    h1 = _tc1(x_pad, aggr1, W1p, b1.reshape(1, HID), W2, b2.reshape(1, HID))
    aggr2 = _sc_aggr64(src2d, dst2d, z64, h1)
    return _tc2(h1, aggr2, batch2d, W3, b3.reshape(1, HID),
                W4, b4.reshape(1, HID))


# speedup vs baseline: 1.8293x; 1.0070x over previous
import functools

import jax
import jax.numpy as jnp
from jax import lax
from jax.experimental import pallas as pl
from jax.experimental.pallas import tpu as pltpu
from jax.experimental.pallas import tpu_sc as plsc

N = 50000
E = 800000
HID = 64
G = 128

NP = 50176
EP = 819200
EROWS = EP // 128
ROWS_PER_TILE = EROWS // 16
HALF = 25000
ACC_ROWS = 25088
TRASH = 25080
TPS = 1568


def _sc_aggr(d_feat):
    mesh = plsc.VectorSubcoreMesh(core_axis_name="c", subcore_axis_name="s")

    @functools.partial(
        pl.kernel,
        out_type=jax.ShapeDtypeStruct((NP, d_feat), jnp.float32),
        mesh=mesh,
        compiler_params=pltpu.CompilerParams(use_tc_tiling_on_sc=False),
        scratch_types=[
            pltpu.VMEM((16, 128), jnp.int32),
            pltpu.VMEM((16, 128), jnp.int32),
            pltpu.VMEM((16, 128), jnp.int32),
            pltpu.VMEM((128, d_feat), jnp.float32),
            pltpu.VMEM_SHARED((ACC_ROWS, d_feat), jnp.float32),
            pltpu.SemaphoreType.DMA,
        ],
    )
    def k(src_hbm, dst_hbm, zeros_hbm, vals_hbm, out_hbm,
          src_v, dst_v, idx_v, rows_v, acc, sem):
        c = lax.axis_index("c")
        s = lax.axis_index("s")
        pltpu.sync_copy(zeros_hbm, acc.at[pl.ds(s * TPS, TPS)])
        plsc.subcore_barrier()
        lo = c * HALF

        def body(i, carry):
            r0 = s * ROWS_PER_TILE + i * 16
            pltpu.sync_copy(src_hbm.at[pl.ds(r0, 16)], src_v)
            pltpu.sync_copy(dst_hbm.at[pl.ds(r0, 16)], dst_v)
            for j in range(16):
                for t in range(8):
                    d = dst_v[j, pl.ds(t * 16, 16)]
                    dl = d - lo
                    ok = (dl >= 0) & (dl < HALF)
                    idx_v[j, pl.ds(t * 16, 16)] = jnp.where(ok, dl, TRASH)
            for j in range(16):
                pltpu.async_copy(vals_hbm.at[src_v.at[j]], rows_v, sem).wait()
                pltpu.sync_copy(rows_v, acc.at[idx_v.at[j]], add=True)
            return carry

        lax.fori_loop(0, ROWS_PER_TILE // 16, body, 0)
        plsc.subcore_barrier()
        @pl.when(s < 15)
        def _():
            pltpu.sync_copy(acc.at[pl.ds(s * TPS, TPS)],
                            out_hbm.at[pl.ds(lo + s * TPS, TPS)])

        @pl.when(s == 15)
        def _():
            pltpu.sync_copy(acc.at[pl.ds(15 * TPS, HALF - 15 * TPS)],
                            out_hbm.at[pl.ds(lo + 15 * TPS, HALF - 15 * TPS)])

    return k


_sc_aggr16 = _sc_aggr(16)
_sc_aggr64 = _sc_aggr(HID)


BM = 1568


def _tc1_body(x_ref, a_ref, w1_ref, b1_ref, w2_ref, b2_ref, o_ref):
    z = x_ref[...] + a_ref[...]
    h = jnp.maximum(
        jnp.dot(z, w1_ref[...], preferred_element_type=jnp.float32)
        + b1_ref[...], 0.0)
    o_ref[...] = jnp.maximum(
        jnp.dot(h, w2_ref[...], preferred_element_type=jnp.float32)
        + b2_ref[...], 0.0)


def _tc1(x_pad, aggr1, W1p, b1, W2, b2):
    grid = NP // BM
    return pl.pallas_call(
        _tc1_body,
        grid=(grid,),
        in_specs=[
            pl.BlockSpec((BM, 16), lambda i: (i, 0)),
            pl.BlockSpec((BM, 16), lambda i: (i, 0)),
            pl.BlockSpec((16, HID), lambda i: (0, 0)),
            pl.BlockSpec((1, HID), lambda i: (0, 0)),
            pl.BlockSpec((HID, HID), lambda i: (0, 0)),
            pl.BlockSpec((1, HID), lambda i: (0, 0)),
        ],
        out_specs=pl.BlockSpec((BM, HID), lambda i: (i, 0)),
        out_shape=jax.ShapeDtypeStruct((NP, HID), jnp.float32),
        compiler_params=pltpu.CompilerParams(
            dimension_semantics=("arbitrary",)),
    )(x_pad, aggr1, W1p, b1, W2, b2)


def _tc2_body(h_ref, a_ref, b_ref, w3_ref, b3_ref, w4_ref, b4_ref,
              pool_ref, out_ref):
    i = pl.program_id(0)
    nblk = pl.num_programs(0)
    z = h_ref[...] + a_ref[...]
    t = jnp.maximum(
        jnp.dot(z, w3_ref[...], preferred_element_type=jnp.float32)
        + b3_ref[...], 0.0)
    h2 = jnp.maximum(
        jnp.dot(t, w4_ref[...], preferred_element_type=jnp.float32)
        + b4_ref[...], 0.0)
    bidx = b_ref[...]
    valid = bidx < G
    h2 = jnp.where(valid, h2, 0.0)
    onehot = (bidx == lax.broadcasted_iota(jnp.int32, (BM, G), 1))
    onehot = onehot.astype(jnp.float32)
    ones_col = jnp.where(valid, 1.0, 0.0)
    hc = jnp.concatenate(
        [h2, ones_col, jnp.zeros((BM, 15), jnp.float32)], axis=1)
    contrib = lax.dot_general(
        onehot, hc, (((0,), (0,)), ((), ())),
        preferred_element_type=jnp.float32)

    @pl.when(i == 0)
    def _():
        pool_ref[...] = jnp.zeros_like(pool_ref)

    pool_ref[...] += contrib

    @pl.when(i == nblk - 1)
    def _():
        p = pool_ref[...]
        cnt = jnp.maximum(p[:, HID:HID + 1], 1.0)
        out_ref[...] = p[:, :HID] / cnt


def _tc2(h1, aggr2, batch2d, W3, b3, W4, b4):
    grid = NP // BM
    _, out = pl.pallas_call(
        _tc2_body,
        grid=(grid,),
        in_specs=[
            pl.BlockSpec((BM, HID), lambda i: (i, 0)),
            pl.BlockSpec((BM, HID), lambda i: (i, 0)),
            pl.BlockSpec((BM, 1), lambda i: (i, 0)),
            pl.BlockSpec((HID, HID), lambda i: (0, 0)),
            pl.BlockSpec((1, HID), lambda i: (0, 0)),
            pl.BlockSpec((HID, HID), lambda i: (0, 0)),
            pl.BlockSpec((1, HID), lambda i: (0, 0)),
        ],
        out_specs=[
            pl.BlockSpec((G, HID + 16), lambda i: (0, 0)),
            pl.BlockSpec((G, HID), lambda i: (0, 0)),
        ],
        out_shape=[
            jax.ShapeDtypeStruct((G, HID + 16), jnp.float32),
            jax.ShapeDtypeStruct((G, HID), jnp.float32),
        ],
        compiler_params=pltpu.CompilerParams(
            dimension_semantics=("arbitrary",)),
    )(h1, aggr2, batch2d, W3, b3, W4, b4)
    return out


def kernel(x, edge_index, batch, W1, b1, W2, b2, W3, b3, W4, b4):
    ei = jnp.asarray(edge_index, jnp.int32)
    src = jnp.concatenate([ei[0], jnp.zeros((EP - E,), jnp.int32)])
    dst = jnp.concatenate([ei[1], jnp.full((EP - E,), 1 << 20, jnp.int32)])
    src2d = src.reshape(EROWS, 128)
    dst2d = dst.reshape(EROWS, 128)

    x_pad = jnp.pad(x, ((0, NP - N), (0, 16 - x.shape[1])))
    W1p = jnp.pad(W1, ((0, 16 - W1.shape[0]), (0, 0)))
    batch2d = jnp.concatenate(
        [jnp.asarray(batch, jnp.int32),
         jnp.full((NP - N,), G, jnp.int32)]).reshape(NP, 1)

    z16 = jnp.zeros((TPS, 16), jnp.float32)
    z64 = jnp.zeros((TPS, HID), jnp.float32)

    aggr1 = _sc_aggr16(src2d, dst2d, z16, x_pad)
    h1 = _tc1(x_pad, aggr1, W1p, b1.reshape(1, HID), W2, b2.reshape(1, HID))
    aggr2 = _sc_aggr64(src2d, dst2d, z64, h1)
    return _tc2(h1, aggr2, batch2d, W3, b3.reshape(1, HID),
                W4, b4.reshape(1, HID))
